# Initial kernel scaffold; baseline (speedup 1.0000x reference)
import jax
import jax.numpy as jnp
from jax.experimental import pallas as pl


def kernel(y_pred, y_true):
    raise NotImplementedError("write your pallas kernel here")


# speedup vs baseline: 1.3694x; 1.3694x over previous
import functools

import jax
import jax.numpy as jnp
from jax.experimental import pallas as pl

EPS = 1e-09


def _logmse_block(y_pred_ref, y_true_ref, out_ref, *, nblocks, inv_n):
    i = pl.program_id(0)

    yp = y_pred_ref[...]
    yt = y_true_ref[...]
    lp = jnp.log2(jnp.maximum(yp, EPS))
    lt = jnp.log2(jnp.maximum(yt, EPS))
    d = lt - lp
    partial = jnp.sum(d * d).reshape(1, 1)

    @pl.when(i == 0)
    def _init():
        out_ref[...] = partial

    @pl.when(i > 0)
    def _acc():
        out_ref[...] = out_ref[...] + partial

    @pl.when(i == nblocks - 1)
    def _finish():
        out_ref[...] = out_ref[...] * inv_n


@functools.partial(jax.jit, static_argnames=())
def _logmse(y_pred2d, y_true):
    b, s = y_pred2d.shape
    rows = 1024
    nblocks = b // rows
    inv_n = 1.0 / float(b * s)
    out = pl.pallas_call(
        functools.partial(_logmse_block, nblocks=nblocks, inv_n=inv_n),
        grid=(nblocks,),
        in_specs=[
            pl.BlockSpec((rows, s), lambda i: (i, 0)),
            pl.BlockSpec((rows, 1), lambda i: (i, 0)),
        ],
        out_specs=pl.BlockSpec((1, 1), lambda i: (0, 0)),
        out_shape=jax.ShapeDtypeStruct((1, 1), jnp.float32),
    )(y_pred2d, y_true)
    return out[0, 0]


def kernel(y_pred, y_true):
    b, s, _ = y_pred.shape
    loss = _logmse(y_pred.reshape(b, s), y_true)
    return (loss, loss, jnp.array(0.0, dtype=jnp.float32))


# speedup vs baseline: 1.8299x; 1.3363x over previous
import functools

import jax
import jax.numpy as jnp
from jax.experimental import pallas as pl

EPS = 1e-09


def _logmse_block(y_pred_ref, y_true_ref, out_ref, *, nblocks, inv_n):
    i = pl.program_id(0)

    yp = y_pred_ref[...]
    yt = y_true_ref[...]
    lp = jnp.log2(jnp.maximum(yp, EPS))
    lt = jnp.log2(jnp.maximum(yt, EPS))
    d = lt - lp
    partial = jnp.sum(d * d).reshape(1, 1)

    @pl.when(i == 0)
    def _init():
        out_ref[...] = partial

    @pl.when(i > 0)
    def _acc():
        out_ref[...] = out_ref[...] + partial

    @pl.when(i == nblocks - 1)
    def _finish():
        out_ref[...] = out_ref[...] * inv_n


def kernel(y_pred, y_true):
    b, s, _ = y_pred.shape
    lanes = 128
    reps = s // lanes
    n = b * reps
    yp = y_pred.reshape(n, lanes)
    yt = jnp.broadcast_to(y_true.reshape(b, 1, 1), (b, reps, 1)).reshape(n, 1)
    rows = 4096
    nblocks = n // rows
    inv_n = 1.0 / float(b * s)
    out = pl.pallas_call(
        functools.partial(_logmse_block, nblocks=nblocks, inv_n=inv_n),
        grid=(nblocks,),
        in_specs=[
            pl.BlockSpec((rows, lanes), lambda i: (i, 0)),
            pl.BlockSpec((rows, 1), lambda i: (i, 0)),
        ],
        out_specs=pl.BlockSpec((1, 1), lambda i: (0, 0)),
        out_shape=jax.ShapeDtypeStruct((1, 1), jnp.float32),
    )(yp, yt)
    loss = out[0, 0]
    return (loss, loss, jnp.array(0.0, dtype=jnp.float32))
